# baseline (speedup 1.0000x reference)
import functools

import jax
import jax.numpy as jnp
from jax import lax
from jax.experimental import pallas as pl
from jax.experimental.pallas import tpu as pltpu
from jax.experimental.pallas import tpu_sc as plsc

N = 16384
DIM = 32
PACK = 128 // DIM
WROWS = 1000000 // PACK
REG_USER = 0.0025
REG_POS_ITEM = 0.0025
REG_NEG_ITEM = 0.00025
REG_BIAS = 0.001

_INFO = plsc.get_sparse_core_info()
_NC = _INFO.num_cores
_NS = _INFO.num_subcores
_NW = _NC * _NS
_BPW = N // _NW
_CHUNK = 256
_NCHUNK = _BPW // _CHUNK
_L = 16


def _sc_body(u_hbm, i_hbm, j_hbm, w_hbm, h_hbm, b_hbm,
             x_hbm, swu_hbm, shi_hbm, shj_hbm, bs_hbm,
             u_v, i_v, j_v, uq_v, iq_v, jq_v, wu_v, hi_v, hj_v, bi_v, bj_v,
             x_v, swu_v, shi_v, shj_v, bs_v, sem):
    wid = lax.axis_index("s") * _NC + lax.axis_index("c")
    base = wid * _BPW

    pltpu.sync_copy(u_hbm.at[pl.ds(base, _BPW)], u_v)
    pltpu.sync_copy(i_hbm.at[pl.ds(base, _BPW)], i_v)
    pltpu.sync_copy(j_hbm.at[pl.ds(base, _BPW)], j_v)

    for s in range(_BPW // _L):
        sl = pl.ds(s * _L, _L)
        uq_v[sl] = jnp.right_shift(u_v[sl], 2)
        iq_v[sl] = jnp.right_shift(i_v[sl], 2)
        jq_v[sl] = jnp.right_shift(j_v[sl], 2)

    cb1 = pltpu.async_copy(b_hbm.at[i_v], bi_v, sem)
    cb2 = pltpu.async_copy(b_hbm.at[j_v], bj_v, sem)

    iota = lax.iota(jnp.int32, _L)

    for ch in range(_NCHUNK):
        c0 = ch * _CHUNK
        c1 = pltpu.async_copy(w_hbm.at[uq_v.at[pl.ds(c0, _CHUNK)]], wu_v, sem)
        c2 = pltpu.async_copy(h_hbm.at[iq_v.at[pl.ds(c0, _CHUNK)]], hi_v, sem)
        c3 = pltpu.async_copy(h_hbm.at[jq_v.at[pl.ds(c0, _CHUNK)]], hj_v, sem)
        c1.wait()
        c2.wait()
        c3.wait()

        def block(blk, _):
            r0 = blk * _L
            rows = r0 + iota
            g0 = c0 + r0
            uoff = jnp.left_shift(jnp.bitwise_and(u_v[pl.ds(g0, _L)], PACK - 1), 5)
            ioff = jnp.left_shift(jnp.bitwise_and(i_v[pl.ds(g0, _L)], PACK - 1), 5)
            joff = jnp.left_shift(jnp.bitwise_and(j_v[pl.ds(g0, _L)], PACK - 1), 5)
            zero = jnp.zeros((_L,), jnp.float32)
            acc_ui = zero
            acc_uj = zero
            acc_wu = zero
            acc_hi = zero
            acc_hj = zero
            for d in range(DIM):
                cw = plsc.load_gather(wu_v, [rows, uoff + d])
                ci = plsc.load_gather(hi_v, [rows, ioff + d])
                cj = plsc.load_gather(hj_v, [rows, joff + d])
                acc_ui = acc_ui + cw * ci
                acc_uj = acc_uj + cw * cj
                acc_wu = acc_wu + cw * cw
                acc_hi = acc_hi + ci * ci
                acc_hj = acc_hj + cj * cj
            bi = bi_v[pl.ds(g0, _L)]
            bj = bj_v[pl.ds(g0, _L)]
            x_v[pl.ds(g0, _L)] = acc_ui - acc_uj + bi - bj
            swu_v[pl.ds(g0, _L)] = acc_wu
            shi_v[pl.ds(g0, _L)] = acc_hi
            shj_v[pl.ds(g0, _L)] = acc_hj
            bs_v[pl.ds(g0, _L)] = bi + bj
            return 0

        if ch == 0:
            cb1.wait()
            cb2.wait()
        lax.fori_loop(0, _CHUNK // _L, block, 0)

    pltpu.sync_copy(x_v, x_hbm.at[pl.ds(base, _BPW)])
    pltpu.sync_copy(swu_v, swu_hbm.at[pl.ds(base, _BPW)])
    pltpu.sync_copy(shi_v, shi_hbm.at[pl.ds(base, _BPW)])
    pltpu.sync_copy(shj_v, shj_hbm.at[pl.ds(base, _BPW)])
    pltpu.sync_copy(bs_v, bs_hbm.at[pl.ds(base, _BPW)])


@jax.jit
def _sc_partials(u, i, j, W, H, B):
    f32 = jnp.float32
    Ww = W.reshape(WROWS, PACK * DIM)
    Hw = H.reshape(WROWS, PACK * DIM)
    mesh = plsc.VectorSubcoreMesh(core_axis_name="c", subcore_axis_name="s")
    out = pl.kernel(
        _sc_body,
        mesh=mesh,
        compiler_params=pltpu.CompilerParams(needs_layout_passes=False),
        out_type=[jax.ShapeDtypeStruct((N,), f32) for _ in range(5)],
        scratch_types=[
            pltpu.VMEM((_BPW,), jnp.int32),
            pltpu.VMEM((_BPW,), jnp.int32),
            pltpu.VMEM((_BPW,), jnp.int32),
            pltpu.VMEM((_BPW,), jnp.int32),
            pltpu.VMEM((_BPW,), jnp.int32),
            pltpu.VMEM((_BPW,), jnp.int32),
            pltpu.VMEM((_CHUNK, PACK * DIM), f32),
            pltpu.VMEM((_CHUNK, PACK * DIM), f32),
            pltpu.VMEM((_CHUNK, PACK * DIM), f32),
            pltpu.VMEM((_BPW,), f32),
            pltpu.VMEM((_BPW,), f32),
            pltpu.VMEM((_BPW,), f32),
            pltpu.VMEM((_BPW,), f32),
            pltpu.VMEM((_BPW,), f32),
            pltpu.VMEM((_BPW,), f32),
            pltpu.VMEM((_BPW,), f32),
            pltpu.SemaphoreType.DMA,
        ],
    )(u, i, j, Ww, Hw, B)
    return out


def _tc_body(x_ref, swu_ref, shi_ref, shj_ref, bs_ref, out_ref):
    x = x_ref[...]
    lp = jnp.mean(-jnp.log(1.0 + jnp.exp(-x)))
    lp = lp - REG_USER * jnp.mean(jnp.sqrt(swu_ref[...]))
    lp = lp - REG_POS_ITEM * jnp.mean(jnp.sqrt(shi_ref[...]))
    lp = lp - REG_NEG_ITEM * jnp.mean(jnp.sqrt(shj_ref[...]))
    lp = lp - REG_BIAS * jnp.mean(bs_ref[...])
    out_ref[0, 0] = -lp


@jax.jit
def _tc_reduce(x, swu, shi, shj, bs):
    r = lambda a: a.reshape(128, 128)
    out = pl.pallas_call(
        _tc_body,
        out_shape=jax.ShapeDtypeStruct((1, 1), jnp.float32),
        out_specs=pl.BlockSpec(memory_space=pltpu.SMEM),
    )(r(x), r(swu), r(shi), r(shj), r(bs))
    return out[0, 0]


def kernel(u, i, j, W, H, B):
    x, swu, shi, shj, bs = _sc_partials(u, i, j, W, H, B)
    return _tc_reduce(x, swu, shi, shj, bs)


# speedup vs baseline: 1.4510x; 1.4510x over previous
import functools

import jax
import jax.numpy as jnp
from jax import lax
from jax.experimental import pallas as pl
from jax.experimental.pallas import tpu as pltpu
from jax.experimental.pallas import tpu_sc as plsc

N = 16384
DIM = 32
PACK = 128 // DIM
WROWS = 1000000 // PACK
REG_USER = 0.0025
REG_POS_ITEM = 0.0025
REG_NEG_ITEM = 0.00025
REG_BIAS = 0.001

_INFO = plsc.get_sparse_core_info()
_NC = _INFO.num_cores
_NS = _INFO.num_subcores
_NW = _NC * _NS
_BPW = N // _NW
_CHUNK = 256
_NCHUNK = _BPW // _CHUNK
_L = 16


def _sc_body(u_hbm, i_hbm, j_hbm, w_hbm, h_hbm, b_hbm,
             x_hbm, swu_hbm, shi_hbm, shj_hbm, bs_hbm,
             u_v, i_v, j_v, uq_v, iq_v, jq_v, wu_v, hi_v, hj_v, bi_v, bj_v,
             x_v, swu_v, shi_v, shj_v, bs_v, sem):
    wid = lax.axis_index("s") * _NC + lax.axis_index("c")
    base = wid * _BPW

    pltpu.sync_copy(u_hbm.at[pl.ds(base, _BPW)], u_v)
    pltpu.sync_copy(i_hbm.at[pl.ds(base, _BPW)], i_v)
    pltpu.sync_copy(j_hbm.at[pl.ds(base, _BPW)], j_v)

    def srow(v):
        return jnp.bitwise_or(
            jnp.left_shift(jnp.right_shift(v, 11), 9),
            jnp.bitwise_and(v, _TSUB - 1),
        )

    for s in range(_BPW // _L):
        sl = pl.ds(s * _L, _L)
        uq_v[sl] = srow(u_v[sl])
        iq_v[sl] = srow(i_v[sl])
        jq_v[sl] = srow(j_v[sl])

    cb1 = pltpu.async_copy(b_hbm.at[i_v], bi_v, sem)
    cb2 = pltpu.async_copy(b_hbm.at[j_v], bj_v, sem)

    iota = lax.iota(jnp.int32, _L)

    for ch in range(_NCHUNK):
        c0 = ch * _CHUNK
        c1 = pltpu.async_copy(w_hbm.at[uq_v.at[pl.ds(c0, _CHUNK)]], wu_v, sem)
        c2 = pltpu.async_copy(h_hbm.at[iq_v.at[pl.ds(c0, _CHUNK)]], hi_v, sem)
        c3 = pltpu.async_copy(h_hbm.at[jq_v.at[pl.ds(c0, _CHUNK)]], hj_v, sem)
        c1.wait()
        c2.wait()
        c3.wait()

        def block(blk, _):
            r0 = blk * _L
            rows = r0 + iota
            g0 = c0 + r0
            def soff(v):
                return jnp.left_shift(
                    jnp.bitwise_and(jnp.right_shift(v, 9), PACK - 1), 5
                )

            uoff = soff(u_v[pl.ds(g0, _L)])
            ioff = soff(i_v[pl.ds(g0, _L)])
            joff = soff(j_v[pl.ds(g0, _L)])
            zero = jnp.zeros((_L,), jnp.float32)
            acc_ui = zero
            acc_uj = zero
            acc_wu = zero
            acc_hi = zero
            acc_hj = zero
            for d in range(DIM):
                cw = plsc.load_gather(wu_v, [rows, uoff + d])
                ci = plsc.load_gather(hi_v, [rows, ioff + d])
                cj = plsc.load_gather(hj_v, [rows, joff + d])
                acc_ui = acc_ui + cw * ci
                acc_uj = acc_uj + cw * cj
                acc_wu = acc_wu + cw * cw
                acc_hi = acc_hi + ci * ci
                acc_hj = acc_hj + cj * cj
            bi = bi_v[pl.ds(g0, _L)]
            bj = bj_v[pl.ds(g0, _L)]
            x_v[pl.ds(g0, _L)] = acc_ui - acc_uj + bi - bj
            swu_v[pl.ds(g0, _L)] = acc_wu
            shi_v[pl.ds(g0, _L)] = acc_hi
            shj_v[pl.ds(g0, _L)] = acc_hj
            bs_v[pl.ds(g0, _L)] = bi + bj
            return 0

        if ch == 0:
            cb1.wait()
            cb2.wait()
        lax.fori_loop(0, _CHUNK // _L, block, 0)

    pltpu.sync_copy(x_v, x_hbm.at[pl.ds(base, _BPW)])
    pltpu.sync_copy(swu_v, swu_hbm.at[pl.ds(base, _BPW)])
    pltpu.sync_copy(shi_v, shi_hbm.at[pl.ds(base, _BPW)])
    pltpu.sync_copy(shj_v, shj_hbm.at[pl.ds(base, _BPW)])
    pltpu.sync_copy(bs_v, bs_hbm.at[pl.ds(base, _BPW)])


_TCOLS = 2048
_TSUB = _TCOLS // PACK
_TGRID = (1000000 + _TCOLS - 1) // _TCOLS
_QROWS = _TGRID * _TSUB


def _pack_block(x):
    y = x.T
    return jnp.concatenate(
        [y[p * _TSUB:(p + 1) * _TSUB, :] for p in range(PACK)], axis=1
    )


def _reformat_body(wt_ref, ht_ref, wo_ref, ho_ref):
    wo_ref[...] = _pack_block(wt_ref[...])
    ho_ref[...] = _pack_block(ht_ref[...])


def _reformat(W, H):
    f32 = jnp.float32
    spec_in = pl.BlockSpec((DIM, _TCOLS), lambda g: (0, g))
    spec_out = pl.BlockSpec((_TSUB, PACK * DIM), lambda g: (g, 0))
    return pl.pallas_call(
        _reformat_body,
        grid=(_TGRID,),
        in_specs=[spec_in, spec_in],
        out_specs=[spec_out, spec_out],
        out_shape=[jax.ShapeDtypeStruct((_QROWS, PACK * DIM), f32)] * 2,
    )(W.T, H.T)


@jax.jit
def _sc_partials(u, i, j, W, H, B):
    f32 = jnp.float32
    Ww, Hw = _reformat(W, H)
    mesh = plsc.VectorSubcoreMesh(core_axis_name="c", subcore_axis_name="s")
    out = pl.kernel(
        _sc_body,
        mesh=mesh,
        compiler_params=pltpu.CompilerParams(needs_layout_passes=False),
        out_type=[jax.ShapeDtypeStruct((N,), f32) for _ in range(5)],
        scratch_types=[
            pltpu.VMEM((_BPW,), jnp.int32),
            pltpu.VMEM((_BPW,), jnp.int32),
            pltpu.VMEM((_BPW,), jnp.int32),
            pltpu.VMEM((_BPW,), jnp.int32),
            pltpu.VMEM((_BPW,), jnp.int32),
            pltpu.VMEM((_BPW,), jnp.int32),
            pltpu.VMEM((_CHUNK, PACK * DIM), f32),
            pltpu.VMEM((_CHUNK, PACK * DIM), f32),
            pltpu.VMEM((_CHUNK, PACK * DIM), f32),
            pltpu.VMEM((_BPW,), f32),
            pltpu.VMEM((_BPW,), f32),
            pltpu.VMEM((_BPW,), f32),
            pltpu.VMEM((_BPW,), f32),
            pltpu.VMEM((_BPW,), f32),
            pltpu.VMEM((_BPW,), f32),
            pltpu.VMEM((_BPW,), f32),
            pltpu.SemaphoreType.DMA,
        ],
    )(u, i, j, Ww, Hw, B)
    return out


def _tc_body(x_ref, swu_ref, shi_ref, shj_ref, bs_ref, out_ref):
    x = x_ref[...]
    lp = jnp.mean(-jnp.log(1.0 + jnp.exp(-x)))
    lp = lp - REG_USER * jnp.mean(jnp.sqrt(swu_ref[...]))
    lp = lp - REG_POS_ITEM * jnp.mean(jnp.sqrt(shi_ref[...]))
    lp = lp - REG_NEG_ITEM * jnp.mean(jnp.sqrt(shj_ref[...]))
    lp = lp - REG_BIAS * jnp.mean(bs_ref[...])
    out_ref[0, 0] = -lp


@jax.jit
def _tc_reduce(x, swu, shi, shj, bs):
    r = lambda a: a.reshape(128, 128)
    out = pl.pallas_call(
        _tc_body,
        out_shape=jax.ShapeDtypeStruct((1, 1), jnp.float32),
        out_specs=pl.BlockSpec(memory_space=pltpu.SMEM),
    )(r(x), r(swu), r(shi), r(shj), r(bs))
    return out[0, 0]


def kernel(u, i, j, W, H, B):
    x, swu, shi, shj, bs = _sc_partials(u, i, j, W, H, B)
    return _tc_reduce(x, swu, shi, shj, bs)
